# baseline (speedup 1.0000x reference)
import functools

import jax
import jax.numpy as jnp
from jax import lax
from jax.experimental import pallas as pl
from jax.experimental.pallas import tpu as pltpu
from jax.experimental.pallas import tpu_sc as plsc

B = 16384
D = 128
NC = 2
NS = 16
NW = NC * NS
BPW = B // NW
NCHUNK = 4
CHUNK = BPW // NCHUNK
L = 16


def _fused_body(hand_ref, act_ref, out_ref):
    r = lax.broadcasted_iota(jnp.int32, (32, 1), 0)
    hsel = (r // 6 == lax.broadcasted_iota(jnp.int32, (32, 5), 1)).astype(jnp.float32)
    asel = (r % 6 == lax.broadcasted_iota(jnp.int32, (32, 6), 1)).astype(jnp.float32)
    hand = jnp.dot(hsel, hand_ref[...], preferred_element_type=jnp.float32,
                   precision=lax.Precision.HIGHEST)
    act = jnp.dot(asel, act_ref[...], preferred_element_type=jnp.float32,
                  precision=lax.Precision.HIGHEST)
    pad = jnp.zeros((32, 1), jnp.float32)
    out_ref[...] = jnp.concatenate([hand, act, pad], axis=1)


_mesh = plsc.VectorSubcoreMesh(core_axis_name="c", subcore_axis_name="s")


@functools.partial(
    pl.kernel,
    mesh=_mesh,
    out_type=jax.ShapeDtypeStruct((B, D), jnp.float32),
    compiler_params=pltpu.CompilerParams(needs_layout_passes=False),
    scratch_types=[
        pltpu.VMEM((BPW * 3,), jnp.int32),
        pltpu.VMEM((CHUNK,), jnp.int32),
        pltpu.VMEM((CHUNK,), jnp.int32),
        pltpu.VMEM((CHUNK,), jnp.int32),
        pltpu.VMEM((CHUNK,), jnp.int32),
        pltpu.VMEM((BPW,), jnp.float32),
        pltpu.VMEM((BPW, D), jnp.float32),
        pltpu.SemaphoreType.DMA,
        pltpu.SemaphoreType.DMA,
    ],
)
def _gather_kernel(fused_hbm, x_hbm, out_hbm,
                   x_v, i0, i1, i2, i3, bets_v, rows_v, sem_g, sem_o):
    wid = lax.axis_index("s") * NC + lax.axis_index("c")
    base = wid * BPW
    pltpu.sync_copy(x_hbm.at[pl.ds(base * 3, BPW * 3)], x_v)
    lanes = lax.iota(jnp.int32, L)
    idx_bufs = [i0, i1, i2, i3]
    for i in range(BPW // L):
        rows = lanes + (i * L)
        flat = rows * 3
        c0 = plsc.load_gather(x_v, [flat])
        c1 = plsc.load_gather(x_v, [flat + 1])
        c2 = plsc.load_gather(x_v, [flat + 2])
        idx_bufs[i // 8][pl.ds((i % 8) * L, L)] = c0 * 6 + c1
        bets_v[pl.ds(i * L, L)] = c2.astype(jnp.float32)
    gathers = [
        pltpu.async_copy(fused_hbm.at[idx_bufs[j]],
                         rows_v.at[pl.ds(j * CHUNK, CHUNK)], sem_g)
        for j in range(NCHUNK)
    ]
    col_last = jnp.full((L,), D - 1, jnp.int32)
    outs = []
    for j in range(NCHUNK):
        gathers[j].wait()
        for i in range(CHUNK // L):
            rows = lanes + (j * CHUNK + i * L)
            plsc.store_scatter(rows_v, [rows, col_last],
                               bets_v[pl.ds(j * CHUNK + i * L, L)])
        outs.append(pltpu.async_copy(
            rows_v.at[pl.ds(j * CHUNK, CHUNK)],
            out_hbm.at[pl.ds(base + j * CHUNK, CHUNK)], sem_o))
    for o in outs:
        o.wait()


def kernel(x, hand_table, action_table):
    x32 = x.astype(jnp.int32)
    fused = pl.pallas_call(
        _fused_body,
        out_shape=jax.ShapeDtypeStruct((32, D), jnp.float32),
    )(hand_table, action_table)
    return _gather_kernel(fused, x32.reshape(-1))


# speedup vs baseline: 1.7794x; 1.7794x over previous
import functools

import jax
import jax.numpy as jnp
from jax import lax
from jax.experimental import pallas as pl
from jax.experimental.pallas import tpu as pltpu
from jax.experimental.pallas import tpu_sc as plsc

B = 16384
D = 128
NC = 2
NS = 16
NW = NC * NS
BPW = B // NW
NCHUNK = 4
CHUNK = BPW // NCHUNK
L = 16
HN, HD = 5, 64
AN, AD = 6, 63

_mesh = plsc.VectorSubcoreMesh(core_axis_name="c", subcore_axis_name="s")


@functools.partial(
    pl.kernel,
    mesh=_mesh,
    out_type=jax.ShapeDtypeStruct((B, D), jnp.float32),
    compiler_params=pltpu.CompilerParams(needs_layout_passes=False),
    scratch_types=[
        pltpu.VMEM_SHARED((32, D), jnp.float32),
        pltpu.VMEM((32, D), jnp.float32),
        pltpu.VMEM((HN * HD,), jnp.float32),
        pltpu.VMEM((AN * AD,), jnp.float32),
        pltpu.VMEM((BPW * 3,), jnp.int32),
        pltpu.VMEM((CHUNK,), jnp.int32),
        pltpu.VMEM((CHUNK,), jnp.int32),
        pltpu.VMEM((CHUNK,), jnp.int32),
        pltpu.VMEM((CHUNK,), jnp.int32),
        pltpu.VMEM((BPW,), jnp.float32),
        pltpu.VMEM((BPW, D), jnp.float32),
        pltpu.SemaphoreType.DMA,
        pltpu.SemaphoreType.DMA,
    ],
)
def _gather_kernel(hand_hbm, act_hbm, x_hbm, out_hbm,
                   fused_sh, fused_v, hand_v, act_v,
                   x_v, i0, i1, i2, i3, bets_v, rows_v, sem_g, sem_o):
    cid = lax.axis_index("c")
    sid = lax.axis_index("s")
    wid = sid * NC + cid
    base = wid * BPW
    lanes = lax.iota(jnp.int32, L)

    @pl.when(sid == 0)
    def _build():
        pltpu.sync_copy(hand_hbm, hand_v)
        pltpu.sync_copy(act_hbm, act_v)
        for r in range(HN * AN):
            h, a = r // AN, r % AN
            for k in range(HD // L):
                fused_v[r, pl.ds(k * L, L)] = hand_v[pl.ds(h * HD + k * L, L)]
            for k, (src, dst) in enumerate(((0, 0), (16, 16), (32, 32),
                                            (AD - L, AD - L))):
                vals = plsc.load_gather(act_v, [lanes + (a * AD + src)])
                fused_v[r, pl.ds(HD + dst, L)] = vals
        pltpu.sync_copy(fused_v, fused_sh)

    plsc.subcore_barrier()

    pltpu.sync_copy(x_hbm.at[pl.ds(base * 3, BPW * 3)], x_v)
    idx_bufs = [i0, i1, i2, i3]
    for i in range(BPW // L):
        flat = (lanes + i * L) * 3
        c0 = plsc.load_gather(x_v, [flat])
        c1 = plsc.load_gather(x_v, [flat + 1])
        c2 = plsc.load_gather(x_v, [flat + 2])
        idx_bufs[i // 8][pl.ds((i % 8) * L, L)] = c0 * 6 + c1
        bets_v[pl.ds(i * L, L)] = c2.astype(jnp.float32)
    gathers = [
        pltpu.async_copy(fused_sh.at[idx_bufs[j]],
                         rows_v.at[pl.ds(j * CHUNK, CHUNK)], sem_g)
        for j in range(NCHUNK)
    ]
    col_last = jnp.full((L,), D - 1, jnp.int32)
    outs = []
    for j in range(NCHUNK):
        gathers[j].wait()
        for i in range(CHUNK // L):
            rows = lanes + (j * CHUNK + i * L)
            plsc.store_scatter(rows_v, [rows, col_last],
                               bets_v[pl.ds(j * CHUNK + i * L, L)])
        outs.append(pltpu.async_copy(
            rows_v.at[pl.ds(j * CHUNK, CHUNK)],
            out_hbm.at[pl.ds(base + j * CHUNK, CHUNK)], sem_o))
    for o in outs:
        o.wait()


def kernel(x, hand_table, action_table):
    return _gather_kernel(hand_table.reshape(-1), action_table.reshape(-1),
                          x.astype(jnp.int32).reshape(-1))


# speedup vs baseline: 2.1616x; 1.2148x over previous
import functools

import jax
import jax.numpy as jnp
from jax import lax
from jax.experimental import pallas as pl
from jax.experimental.pallas import tpu as pltpu
from jax.experimental.pallas import tpu_sc as plsc

B = 16384
D = 128
NC = 2
NS = 16
NW = NC * NS
BPW = B // NW
NCHUNK = 4
CHUNK = BPW // NCHUNK
L = 16
HN, HD = 5, 64
AN, AD = 6, 63

_mesh = plsc.VectorSubcoreMesh(core_axis_name="c", subcore_axis_name="s")


@functools.partial(
    pl.kernel,
    mesh=_mesh,
    out_type=jax.ShapeDtypeStruct((B, D), jnp.float32),
    compiler_params=pltpu.CompilerParams(needs_layout_passes=False),
    scratch_types=[
        pltpu.VMEM_SHARED((32, D), jnp.float32),
        pltpu.VMEM((32, D), jnp.float32),
        pltpu.VMEM((HN, HD), jnp.float32),
        pltpu.VMEM((AN, AD), jnp.float32),
        pltpu.VMEM((CHUNK, 3), jnp.int32),
        pltpu.VMEM((CHUNK,), jnp.int32),
        pltpu.VMEM((CHUNK,), jnp.int32),
        pltpu.VMEM((CHUNK,), jnp.int32),
        pltpu.VMEM((CHUNK,), jnp.int32),
        pltpu.VMEM((BPW,), jnp.float32),
        pltpu.VMEM((BPW, D), jnp.float32),
        pltpu.SemaphoreType.DMA,
        pltpu.SemaphoreType.DMA,
    ],
)
def _gather_kernel(hand_hbm, act_hbm, x_hbm, out_hbm,
                   fused_sh, fused_v, hand_v, act_v,
                   x_v, i0, i1, i2, i3, bets_v, rows_v, sem_g, sem_o):
    cid = lax.axis_index("c")
    sid = lax.axis_index("s")
    wid = sid * NC + cid
    base = wid * BPW
    lanes = lax.iota(jnp.int32, L)

    @pl.when(sid == 0)
    def _build():
        pltpu.sync_copy(hand_hbm, hand_v)
        pltpu.sync_copy(act_hbm, act_v)
        for r in range(HN * AN):
            h, a = r // AN, r % AN
            for k in range(HD // L):
                fused_v[r, pl.ds(k * L, L)] = hand_v[h, pl.ds(k * L, L)]
            for src in (0, 16, 32, AD - L):
                fused_v[r, pl.ds(HD + src, L)] = act_v[a, pl.ds(src, L)]
        pltpu.sync_copy(fused_v, fused_sh)

    plsc.subcore_barrier()

    idx_bufs = [i0, i1, i2, i3]
    zero_c = jnp.zeros((L,), jnp.int32)
    gathers = []
    for j in range(NCHUNK):
        pltpu.sync_copy(x_hbm.at[pl.ds(base + j * CHUNK, CHUNK)], x_v)
        for i in range(CHUNK // L):
            rows = lanes + i * L
            c0 = plsc.load_gather(x_v, [rows, zero_c])
            c1 = plsc.load_gather(x_v, [rows, zero_c + 1])
            c2 = plsc.load_gather(x_v, [rows, zero_c + 2])
            idx_bufs[j][pl.ds(i * L, L)] = c0 * 6 + c1
            bets_v[pl.ds(j * CHUNK + i * L, L)] = c2.astype(jnp.float32)
        gathers.append(pltpu.async_copy(
            fused_sh.at[idx_bufs[j]],
            rows_v.at[pl.ds(j * CHUNK, CHUNK)], sem_g))
    col_last = jnp.full((L,), D - 1, jnp.int32)
    outs = []
    for j in range(NCHUNK):
        gathers[j].wait()
        for i in range(CHUNK // L):
            rows = lanes + (j * CHUNK + i * L)
            plsc.store_scatter(rows_v, [rows, col_last],
                               bets_v[pl.ds(j * CHUNK + i * L, L)])
        outs.append(pltpu.async_copy(
            rows_v.at[pl.ds(j * CHUNK, CHUNK)],
            out_hbm.at[pl.ds(base + j * CHUNK, CHUNK)], sem_o))
    for o in outs:
        o.wait()


def kernel(x, hand_table, action_table):
    return _gather_kernel(hand_table, action_table, x.astype(jnp.int32))
